# Initial kernel scaffold; baseline (speedup 1.0000x reference)
import jax
import jax.numpy as jnp
from jax.experimental import pallas as pl


def kernel(news, news_mask, table):
    raise NotImplementedError("write your pallas kernel here")


# speedup vs baseline: 3.5627x; 3.5627x over previous
import functools

import jax
import jax.numpy as jnp
from jax import lax
from jax.experimental import pallas as pl
from jax.experimental.pallas import tpu as pltpu
from jax.experimental.pallas import tpu_sc as plsc

VOCAB = 1000
EMBED_DIM = 64
B = 4096
L = 200

N = B * L
NUM_WORKERS = 32
PER_WORKER = N // NUM_WORKERS
GROUP = 128
SUPER = 512
CHUNKS = SUPER // GROUP
NUM_GROUPS = PER_WORKER // GROUP
NUM_SUPER = PER_WORKER // SUPER


_mesh = plsc.VectorSubcoreMesh(core_axis_name="c", subcore_axis_name="s")


@functools.partial(
    pl.kernel,
    mesh=_mesh,
    out_type=jax.ShapeDtypeStruct((N, EMBED_DIM), jnp.float32),
    scratch_types=[
        pltpu.VMEM((NUM_GROUPS, GROUP), jnp.int32),
        pltpu.VMEM((SUPER, EMBED_DIM), jnp.float32),
        pltpu.VMEM((SUPER, EMBED_DIM), jnp.float32),
        pltpu.SemaphoreType.DMA,
        pltpu.SemaphoreType.DMA,
        pltpu.SemaphoreType.DMA,
        pltpu.SemaphoreType.DMA,
    ],
    compiler_params=pltpu.CompilerParams(use_tc_tiling_on_sc=False),
)
def _embed_sc(news_hbm, table_hbm, out_hbm, idx_v, rows0, rows1,
              gsem0, gsem1, ssem0, ssem1):
    wid = lax.axis_index("s") * 2 + lax.axis_index("c")
    base = wid * PER_WORKER

    pltpu.sync_copy(news_hbm.at[wid], idx_v)

    rows = (rows0, rows1)
    gsems = (gsem0, gsem1)
    ssems = (ssem0, ssem1)

    def start_gathers(s, buf, gsem):
        for c in range(CHUNKS):
            pltpu.async_copy(
                table_hbm.at[idx_v.at[s * CHUNKS + c]],
                buf.at[pl.ds(c * GROUP, GROUP)],
                gsem,
            )

    def wait_gathers(buf, gsem):
        pltpu.make_async_copy(table_hbm.at[pl.ds(0, SUPER)], buf, gsem).wait()

    def wait_store(buf, ssem):
        pltpu.make_async_copy(buf, out_hbm.at[pl.ds(base, SUPER)], ssem).wait()

    start_gathers(0, rows0, gsem0)

    def body(i, _):
        for par in range(2):
            s = i * 2 + par
            b = par
            nb = 1 - par

            @pl.when(s >= 1)
            def _():
                wait_store(rows[nb], ssems[nb])

            @pl.when(s + 1 < NUM_SUPER)
            def _():
                start_gathers(s + 1, rows[nb], gsems[nb])

            wait_gathers(rows[b], gsems[b])
            pltpu.async_copy(
                rows[b], out_hbm.at[pl.ds(base + s * SUPER, SUPER)], ssems[b])
        return 0

    lax.fori_loop(0, NUM_SUPER // 2, body, 0)

    wait_store(rows1, ssem1)


def kernel(news, news_mask, table):
    del news_mask
    idx = news.reshape(NUM_WORKERS, NUM_GROUPS, GROUP)
    out = _embed_sc(idx, table)
    return out.reshape(B, L, EMBED_DIM)


# speedup vs baseline: 5.0134x; 1.4072x over previous
import functools

import jax
import jax.numpy as jnp
from jax import lax
from jax.experimental import pallas as pl
from jax.experimental.pallas import tpu as pltpu
from jax.experimental.pallas import tpu_sc as plsc

VOCAB = 1000
EMBED_DIM = 64
B = 4096
L = 200

N = B * L
NUM_WORKERS = 32
PER_WORKER = N // NUM_WORKERS
GROUP = 128
SUPER = 512
CHUNKS = SUPER // GROUP
NUM_GROUPS = PER_WORKER // GROUP
NUM_SUPER = PER_WORKER // SUPER


_mesh = plsc.VectorSubcoreMesh(core_axis_name="c", subcore_axis_name="s")


@functools.partial(
    pl.kernel,
    mesh=_mesh,
    out_type=jax.ShapeDtypeStruct((N, EMBED_DIM), jnp.float32),
    scratch_types=[
        pltpu.VMEM_SHARED((VOCAB, EMBED_DIM), jnp.float32),
        pltpu.VMEM((NUM_GROUPS, GROUP), jnp.int32),
        pltpu.VMEM((SUPER, EMBED_DIM), jnp.float32),
        pltpu.VMEM((SUPER, EMBED_DIM), jnp.float32),
        pltpu.SemaphoreType.DMA,
        pltpu.SemaphoreType.DMA,
        pltpu.SemaphoreType.DMA,
        pltpu.SemaphoreType.DMA,
    ],
    compiler_params=pltpu.CompilerParams(use_tc_tiling_on_sc=False),
)
def _embed_sc(news_hbm, table_hbm, out_hbm, table_sh, idx_v, rows0, rows1,
              gsem0, gsem1, ssem0, ssem1):
    sid = lax.axis_index("s")
    wid = sid * 2 + lax.axis_index("c")
    base = wid * PER_WORKER

    @pl.when(sid == 0)
    def _():
        pltpu.sync_copy(table_hbm, table_sh)

    pltpu.sync_copy(news_hbm.at[wid], idx_v)
    plsc.subcore_barrier()

    rows = (rows0, rows1)
    gsems = (gsem0, gsem1)
    ssems = (ssem0, ssem1)

    def start_gathers(s, buf, gsem):
        for c in range(CHUNKS):
            pltpu.async_copy(
                table_sh.at[idx_v.at[s * CHUNKS + c]],
                buf.at[pl.ds(c * GROUP, GROUP)],
                gsem,
            )

    def wait_gathers(buf, gsem):
        pltpu.make_async_copy(table_hbm.at[pl.ds(0, SUPER)], buf, gsem).wait()

    def wait_store(buf, ssem):
        pltpu.make_async_copy(buf, out_hbm.at[pl.ds(base, SUPER)], ssem).wait()

    start_gathers(0, rows0, gsem0)

    def body(i, _):
        for par in range(2):
            s = i * 2 + par
            b = par
            nb = 1 - par

            @pl.when(s >= 1)
            def _():
                wait_store(rows[nb], ssems[nb])

            @pl.when(s + 1 < NUM_SUPER)
            def _():
                start_gathers(s + 1, rows[nb], gsems[nb])

            wait_gathers(rows[b], gsems[b])
            pltpu.async_copy(
                rows[b], out_hbm.at[pl.ds(base + s * SUPER, SUPER)], ssems[b])
        return 0

    lax.fori_loop(0, NUM_SUPER // 2, body, 0)

    wait_store(rows1, ssem1)


def kernel(news, news_mask, table):
    del news_mask
    idx = news.reshape(NUM_WORKERS, NUM_GROUPS, GROUP)
    out = _embed_sc(idx, table)
    return out.reshape(B, L, EMBED_DIM)
